# Initial kernel scaffold; baseline (speedup 1.0000x reference)
import jax
import jax.numpy as jnp
from jax.experimental import pallas as pl


def kernel(x, edge_index, batch, W_feat, Ws, bs, lin_W, lin_b, cls_W, cls_b):
    raise NotImplementedError("write your pallas kernel here")


# speedup vs baseline: 4.9591x; 4.9591x over previous
import functools

import jax
import jax.numpy as jnp
from jax import lax
from jax.experimental import pallas as pl
from jax.experimental.pallas import tpu as pltpu
from jax.experimental.pallas import tpu_sc as plsc

N = 10000
E = 160000
D = 256
H = 512
C = 10
G = 128
EPS = 1e-5
BNB = 1e-4

N_T = 10240
RB = 512
NRB = N_T // RB
NBATCH = 80
E_PAD = 16 * NBATCH * 128
DEG_NB = 40
G_PAD = 256
_PREC = lax.Precision.HIGHEST

@functools.cache
def _vmesh():
    return plsc.VectorSubcoreMesh(core_axis_name="c", subcore_axis_name="s")


def _bn_matmul_relu_body(x_ref, w_ref, o_ref, acc_ref):
    s = pl.program_id(0)

    @pl.when(s == 0)
    def _():
        acc_ref[...] = jnp.zeros_like(acc_ref)

    @pl.when(s < NRB)
    def _():
        xb = x_ref[...]
        rows = lax.broadcasted_iota(jnp.int32, (RB, 1), 0) + s * RB
        xm = jnp.where(rows < N, xb, 0.0)
        acc_ref[0:1, :] += jnp.sum(xm, axis=0, keepdims=True)
        acc_ref[1:2, :] += jnp.sum(xm * xm, axis=0, keepdims=True)

    @pl.when(s >= NRB)
    def _():
        xb = x_ref[...]
        m = acc_ref[0:1, :] / N
        v = acc_ref[1:2, :] / N - m * m
        r = lax.rsqrt(v + EPS)
        xbn = (xb - m) * r + BNB
        y = jnp.dot(xbn, w_ref[...], precision=_PREC,
                    preferred_element_type=jnp.float32)
        o_ref[...] = jnp.maximum(y, 0.0)


def _bn_matmul_relu(x, w):
    din = x.shape[1]
    return pl.pallas_call(
        _bn_matmul_relu_body,
        grid=(2 * NRB,),
        in_specs=[
            pl.BlockSpec((RB, din), lambda s: (jnp.where(s < NRB, s, s - NRB), 0)),
            pl.BlockSpec((din, H), lambda s: (0, 0)),
        ],
        out_specs=pl.BlockSpec((RB, H), lambda s: (jnp.where(s < NRB, 0, s - NRB), 0)),
        out_shape=jax.ShapeDtypeStruct((N_T, H), jnp.float32),
        scratch_shapes=[pltpu.VMEM((2, din), jnp.float32)],
    )(x, w)


def _bn_matmul_scale_body(x_ref, w_ref, dinv_ref, o_ref, acc_ref):
    s = pl.program_id(0)

    @pl.when(s == 0)
    def _():
        acc_ref[...] = jnp.zeros_like(acc_ref)

    @pl.when(s < NRB)
    def _():
        xb = x_ref[...]
        rows = lax.broadcasted_iota(jnp.int32, (RB, 1), 0) + s * RB
        xm = jnp.where(rows < N, xb, 0.0)
        acc_ref[0:1, :] += jnp.sum(xm, axis=0, keepdims=True)
        acc_ref[1:2, :] += jnp.sum(xm * xm, axis=0, keepdims=True)

    @pl.when(s >= NRB)
    def _():
        xb = x_ref[...]
        m = acc_ref[0:1, :] / N
        v = acc_ref[1:2, :] / N - m * m
        r = lax.rsqrt(v + EPS)
        xbn = (xb - m) * r + BNB
        y = jnp.dot(xbn, w_ref[...], precision=_PREC,
                    preferred_element_type=jnp.float32)
        t = y * dinv_ref[:, 0:1]
        for c in range(4):
            o_ref[c] = t[:, c * 128:(c + 1) * 128]


def _bn_matmul_scale(x, w, dinv):
    return pl.pallas_call(
        _bn_matmul_scale_body,
        grid=(2 * NRB,),
        in_specs=[
            pl.BlockSpec((RB, H), lambda s: (jnp.where(s < NRB, s, s - NRB), 0)),
            pl.BlockSpec((H, H), lambda s: (0, 0)),
            pl.BlockSpec((RB, 128), lambda s: (jnp.where(s < NRB, 0, s - NRB), 0)),
        ],
        out_specs=pl.BlockSpec(
            (4, RB, 128), lambda s: (0, jnp.where(s < NRB, 0, s - NRB), 0)),
        out_shape=jax.ShapeDtypeStruct((4, N_T, 128), jnp.float32),
        scratch_shapes=[pltpu.VMEM((2, H), jnp.float32)],
    )(x, w, dinv)


def _combine_body(agg_ref, xwp_ref, dinv_ref, b_ref, o_ref):
    dv = dinv_ref[:, 0:1]
    for c in range(4):
        t = dv * (agg_ref[c] + xwp_ref[c]) + b_ref[0:1, c * 128:(c + 1) * 128]
        o_ref[:, c * 128:(c + 1) * 128] = jnp.maximum(t, 0.0)


def _combine(agg, xwp, dinv, b):
    return pl.pallas_call(
        _combine_body,
        grid=(NRB,),
        in_specs=[
            pl.BlockSpec((4, RB, 128), lambda s: (0, s, 0)),
            pl.BlockSpec((4, RB, 128), lambda s: (0, s, 0)),
            pl.BlockSpec((RB, 128), lambda s: (s, 0)),
            pl.BlockSpec((1, H), lambda s: (0, 0)),
        ],
        out_specs=pl.BlockSpec((RB, H), lambda s: (s, 0)),
        out_shape=jax.ShapeDtypeStruct((N_T, H), jnp.float32),
    )(agg, xwp, dinv, b)


def _dinv_body(dacc_ref, o_ref):
    deg = dacc_ref[0, :, 0:1] + dacc_ref[1, :, 0:1] + 1.0
    o_ref[...] = jnp.broadcast_to(lax.rsqrt(deg), (RB, 128))


def _dinv(dacc):
    return pl.pallas_call(
        _dinv_body,
        grid=(NRB,),
        in_specs=[pl.BlockSpec((2, RB, 128), lambda s: (0, s, 0))],
        out_specs=pl.BlockSpec((RB, 128), lambda s: (s, 0)),
        out_shape=jax.ShapeDtypeStruct((N_T, 128), jnp.float32),
    )(dacc)


def _pool_body(x_ref, b_ref, o_ref, acc_ref):
    s = pl.program_id(0)

    @pl.when(s == 0)
    def _():
        acc_ref[...] = jnp.zeros_like(acc_ref)

    pb = (b_ref[...] == lax.broadcasted_iota(jnp.int32, (1, G), 1)
          ).astype(jnp.float32)
    acc_ref[...] += lax.dot_general(
        pb, x_ref[...], (((0,), (0,)), ((), ())), precision=_PREC,
        preferred_element_type=jnp.float32)

    @pl.when(s == NRB - 1)
    def _():
        o_ref[...] = acc_ref[...]


def _pool_tc(x3, batch2d):
    return pl.pallas_call(
        _pool_body,
        grid=(NRB,),
        in_specs=[
            pl.BlockSpec((RB, H), lambda s: (s, 0)),
            pl.BlockSpec((RB, 1), lambda s: (s, 0)),
        ],
        out_specs=pl.BlockSpec((G, H), lambda s: (0, 0)),
        out_shape=jax.ShapeDtypeStruct((G, H), jnp.float32),
        scratch_shapes=[pltpu.VMEM((G, H), jnp.float32)],
    )(x3, batch2d)


def _head_body(g_ref, lw_ref, lb_ref, cw_ref, cb_ref, o_ref):
    g = g_ref[...]
    m = jnp.mean(g, axis=0, keepdims=True)
    v = jnp.mean((g - m) ** 2, axis=0, keepdims=True)
    g = (g - m) * lax.rsqrt(v + EPS) + BNB
    h = jnp.dot(g, lw_ref[...], precision=_PREC,
                preferred_element_type=jnp.float32) + lb_ref[...]
    h = jnp.maximum(h, 0.0)
    m2 = jnp.mean(h, axis=0, keepdims=True)
    v2 = jnp.mean((h - m2) ** 2, axis=0, keepdims=True)
    h = (h - m2) * lax.rsqrt(v2 + EPS) + BNB
    o = jnp.dot(h, cw_ref[...], precision=_PREC,
                preferred_element_type=jnp.float32) + cb_ref[...]
    mx = jnp.max(o, axis=-1, keepdims=True)
    z = o - mx
    o_ref[...] = z - jnp.log(jnp.sum(jnp.exp(z), axis=-1, keepdims=True))


def _head(gacc, lw, lb, cw, cb):
    return pl.pallas_call(
        _head_body,
        out_shape=jax.ShapeDtypeStruct((G, C), jnp.float32),
    )(gacc, lw, lb, cw, cb)


def _deg_sc(didx, ones128, zeros):
    @functools.partial(
        pl.kernel,
        out_type=jax.ShapeDtypeStruct((2, N_T, 128), jnp.float32),
        mesh=_vmesh(),
        scratch_types=[
            pltpu.VMEM((128,), jnp.int32),
            pltpu.VMEM((128, 128), jnp.float32),
            pltpu.VMEM_SHARED((N_T, 128), jnp.float32),
            pltpu.SemaphoreType.DMA,
        ],
    )
    def deg_kernel(didx_hbm, ones_hbm, zeros_hbm, out_hbm,
                   didx_v, ones_v, acc, sem):
        cid = lax.axis_index("c")
        sid = lax.axis_index("s")
        base = (cid * 16 + sid) * (DEG_NB * 128)
        pltpu.sync_copy(ones_hbm, ones_v)
        pltpu.sync_copy(zeros_hbm, acc.at[pl.ds(sid * 640, 640)])
        plsc.subcore_barrier()

        @pl.loop(0, DEG_NB)
        def _(b):
            pltpu.async_copy(didx_hbm.at[pl.ds(base + b * 128, 128)],
                             didx_v, sem).wait()
            pltpu.sync_copy(ones_v, acc.at[didx_v], add=True)

        plsc.subcore_barrier()
        pltpu.sync_copy(acc.at[pl.ds(sid * 640, 640)],
                        out_hbm.at[cid, pl.ds(sid * 640, 640)])

    return deg_kernel(didx, ones128, zeros)


def _msg_sc(table, gidx, sidx, zeros):
    @functools.partial(
        pl.kernel,
        out_type=jax.ShapeDtypeStruct((4, N_T, 128), jnp.float32),
        mesh=_vmesh(),
        scratch_types=[
            pltpu.VMEM((128,), jnp.int32),
            pltpu.VMEM((128,), jnp.int32),
            pltpu.VMEM((128,), jnp.int32),
            pltpu.VMEM((128,), jnp.int32),
            pltpu.VMEM((128, 128), jnp.float32),
            pltpu.VMEM((128, 128), jnp.float32),
            pltpu.VMEM_SHARED((N_T, 128), jnp.float32),
            pltpu.SemaphoreType.DMA,
            pltpu.SemaphoreType.DMA,
            pltpu.SemaphoreType.DMA,
            pltpu.SemaphoreType.DMA,
        ],
    )
    def msg_kernel(table_hbm, gidx_hbm, sidx_hbm, zeros_hbm, out_hbm,
                   gib0, gib1, sib0, sib1, rows0, rows1, acc,
                   sem0, sem1, isem0, isem1):
        cid = lax.axis_index("c")
        sid = lax.axis_index("s")
        sbase = sid * (NBATCH * 128)
        for ci in range(2):
            c = cid * 2 + ci
            gbase = (c * 16 + sid) * (NBATCH * 128)
            pltpu.async_copy(gidx_hbm.at[pl.ds(gbase, 128)], gib0, isem0)
            pltpu.async_copy(sidx_hbm.at[pl.ds(sbase, 128)], sib0, isem0)
            pltpu.async_copy(gidx_hbm.at[pl.ds(gbase + 128, 128)], gib1, isem1)
            pltpu.async_copy(sidx_hbm.at[pl.ds(sbase + 128, 128)], sib1, isem1)
            pltpu.sync_copy(zeros_hbm, acc.at[pl.ds(sid * 640, 640)])
            plsc.subcore_barrier()
            pltpu.make_async_copy(gidx_hbm.at[pl.ds(gbase, 128)],
                                  gib0, isem0).wait()
            pltpu.make_async_copy(sidx_hbm.at[pl.ds(sbase, 128)],
                                  sib0, isem0).wait()
            pltpu.async_copy(table_hbm.at[gib0], rows0, sem0)

            @pl.loop(0, NBATCH, step=2)
            def _(b):
                pltpu.make_async_copy(gidx_hbm.at[pl.ds(gbase, 128)],
                                      gib1, isem1).wait()
                pltpu.make_async_copy(sidx_hbm.at[pl.ds(sbase, 128)],
                                      sib1, isem1).wait()
                pltpu.async_copy(table_hbm.at[gib1], rows1, sem1)
                pltpu.make_async_copy(table_hbm.at[gib0], rows0, sem0).wait()
                pltpu.sync_copy(rows0, acc.at[sib0], add=True)

                @pl.when(b + 2 < NBATCH)
                def _():
                    off = (b + 2) * 128
                    pltpu.async_copy(
                        gidx_hbm.at[pl.ds(gbase + off, 128)], gib0, isem0)
                    pltpu.async_copy(
                        sidx_hbm.at[pl.ds(sbase + off, 128)], sib0, isem0)

                pltpu.make_async_copy(table_hbm.at[gib1], rows1, sem1).wait()
                pltpu.sync_copy(rows1, acc.at[sib1], add=True)

                @pl.when(b + 2 < NBATCH)
                def _():
                    off = (b + 2) * 128
                    pltpu.make_async_copy(
                        gidx_hbm.at[pl.ds(gbase + off, 128)],
                        gib0, isem0).wait()
                    pltpu.make_async_copy(
                        sidx_hbm.at[pl.ds(sbase + off, 128)],
                        sib0, isem0).wait()
                    pltpu.async_copy(table_hbm.at[gib0], rows0, sem0)

                @pl.when(b + 3 < NBATCH)
                def _():
                    off = (b + 3) * 128
                    pltpu.async_copy(
                        gidx_hbm.at[pl.ds(gbase + off, 128)], gib1, isem1)
                    pltpu.async_copy(
                        sidx_hbm.at[pl.ds(sbase + off, 128)], sib1, isem1)

            plsc.subcore_barrier()
            pltpu.sync_copy(acc.at[pl.ds(sid * 640, 640)],
                            out_hbm.at[c, pl.ds(sid * 640, 640)])

    return msg_kernel(table, gidx, sidx, zeros)


def kernel(x, edge_index, batch, W_feat, Ws, bs, lin_W, lin_b, cls_W, cls_b):
    f32 = jnp.float32
    x = x.astype(f32)
    row = edge_index[0].astype(jnp.int32)
    col = edge_index[1].astype(jnp.int32)
    pad = E_PAD - E

    rowg = jnp.concatenate([row, jnp.zeros((pad,), jnp.int32)])
    gidx = (rowg.reshape(1, E_PAD)
            + (jnp.arange(4, dtype=jnp.int32) * N_T).reshape(4, 1)
            ).reshape(4 * E_PAD)
    sidx = jnp.concatenate([col, jnp.full((pad,), N, jnp.int32)])
    didx = jnp.concatenate([row, jnp.full((pad,), N, jnp.int32)])
    bidx = jnp.concatenate(
        [batch.astype(jnp.int32),
         jnp.full((N_T - N,), G, jnp.int32)]).reshape(N_T, 1)
    zeros640 = jnp.zeros((640, 128), f32)
    ones128 = jnp.ones((128, 128), f32)
    xpad = jnp.pad(x, ((0, N_T - N), (0, 0)))

    dacc = _deg_sc(didx, ones128, zeros640)
    dinv = _dinv(dacc)
    cur = _bn_matmul_relu(xpad, W_feat)
    for i in range(3):
        xwp = _bn_matmul_scale(cur, Ws[i], dinv)
        agg = _msg_sc(xwp.reshape(4 * N_T, 128), gidx, sidx, zeros640)
        cur = _combine(agg, xwp, dinv, bs[i].reshape(1, H))
    g = _pool_tc(cur, bidx)
    return _head(g, lin_W, lin_b.reshape(1, H), cls_W, cls_b.reshape(1, C))
